# Initial kernel scaffold; baseline (speedup 1.0000x reference)
import jax
import jax.numpy as jnp
from jax.experimental import pallas as pl


def kernel(prompt_ids, table, W1, b1, W2, b2):
    raise NotImplementedError("write your pallas kernel here")


# speedup vs baseline: 39.6589x; 39.6589x over previous
import functools
import math

import jax
import jax.numpy as jnp
from jax import lax
from jax.experimental import pallas as pl
from jax.experimental.pallas import tpu as pltpu
from jax.experimental.pallas import tpu_sc as plsc

_B, _L, _V, _D = 1024, 512, 256, 128

_NC, _NS, _LANES = 2, 16, 16
_NW = _NC * _NS
_RPW = _B // _NW
_GROUPS = _RPW // _LANES
_VCHUNKS = _V // _LANES


def _sc_hist_body(ids_hbm, h_hbm, ids_v, hist_v):
    wid = lax.axis_index("s") * _NC + lax.axis_index("c")
    base = wid * _RPW
    pltpu.sync_copy(ids_hbm.at[pl.ds(base * _L, _RPW * _L)], ids_v)

    lane = lax.iota(jnp.int32, 16)
    zeros = jnp.zeros((_LANES,), jnp.float32)
    ones = jnp.ones((_LANES,), jnp.float32)

    def _zero(i, carry):
        hist_v[pl.ds(i * _LANES, _LANES)] = zeros
        return carry

    lax.fori_loop(0, _RPW * _VCHUNKS, _zero, 0)

    for g in range(_GROUPS):
        row = g * _LANES + lane

        def _pos(p, carry):
            ids16 = plsc.load_gather(ids_v, [row * _L + p])
            plsc.addupdate_scatter(hist_v, [row * _V + ids16], ones)
            return carry

        lax.fori_loop(0, _L, _pos, 0)

    pltpu.sync_copy(hist_v, h_hbm.at[pl.ds(base * _V, _RPW * _V)])


_sc_hist = functools.partial(
    pl.kernel,
    mesh=plsc.VectorSubcoreMesh(core_axis_name="c", subcore_axis_name="s"),
    compiler_params=pltpu.CompilerParams(needs_layout_passes=False),
    out_type=jax.ShapeDtypeStruct((_B * _V,), jnp.float32),
    scratch_types=[
        pltpu.VMEM((_RPW * _L,), jnp.int32),
        pltpu.VMEM((_RPW * _V,), jnp.float32),
    ],
)(_sc_hist_body)


def _tc_mlp_body(h_ref, tab_ref, w1_ref, b1_ref, w2_ref, b2_ref, o_ref):
    pooled = jnp.dot(h_ref[...], tab_ref[...],
                     preferred_element_type=jnp.float32) * (1.0 / _L)
    x = jnp.dot(pooled, w1_ref[...],
                preferred_element_type=jnp.float32) + b1_ref[...]
    x = 0.5 * x * (1.0 + lax.erf(x * (1.0 / math.sqrt(2.0))))
    o_ref[...] = jnp.dot(x, w2_ref[...],
                         preferred_element_type=jnp.float32) + b2_ref[...]


def kernel(prompt_ids, table, W1, b1, W2, b2):
    h = _sc_hist(prompt_ids.astype(jnp.int32).reshape(_B * _L)).reshape(_B, _V)
    out = pl.pallas_call(
        _tc_mlp_body,
        out_shape=jax.ShapeDtypeStruct((_B, _D), jnp.float32),
    )(h, table, W1, b1.reshape(1, _D), W2, b2.reshape(1, _D))
    return out


# speedup vs baseline: 40.9447x; 1.0324x over previous
import functools
import math

import jax
import jax.numpy as jnp
from jax import lax
from jax.experimental import pallas as pl
from jax.experimental.pallas import tpu as pltpu
from jax.experimental.pallas import tpu_sc as plsc

_B, _L, _V, _D = 1024, 512, 256, 128

_NC, _NS, _LANES = 2, 16, 16
_NW = _NC * _NS
_RPW = _B // _NW
_GROUPS = _RPW // _LANES
_VCHUNKS = _V // _LANES


def _sc_hist_body(ids_hbm, h_hbm, ids_v, hist_v):
    wid = lax.axis_index("s") * _NC + lax.axis_index("c")
    base = wid * _RPW
    pltpu.sync_copy(ids_hbm.at[pl.ds(base * _L, _RPW * _L)], ids_v)

    lane = lax.iota(jnp.int32, 16)
    zeros = jnp.zeros((_LANES,), jnp.float32)
    ones = jnp.ones((_LANES,), jnp.float32)

    def _zero(i, carry):
        for u in range(8):
            hist_v[pl.ds((i * 8 + u) * _LANES, _LANES)] = zeros
        return carry

    lax.fori_loop(0, _RPW * _VCHUNKS // 8, _zero, 0)

    _U = 4

    def _pos(p, carry):
        for g in range(_GROUPS):
            row = g * _LANES + lane
            gbase = row * _L
            hbase = row * _V
            for u in range(_U):
                pos = p * _U + u
                ids16 = plsc.load_gather(ids_v, [gbase + pos])
                plsc.addupdate_scatter(hist_v, [hbase + ids16], ones)
        return carry

    lax.fori_loop(0, _L // _U, _pos, 0)

    pltpu.sync_copy(hist_v, h_hbm.at[pl.ds(base * _V, _RPW * _V)])


_sc_hist = functools.partial(
    pl.kernel,
    mesh=plsc.VectorSubcoreMesh(core_axis_name="c", subcore_axis_name="s"),
    compiler_params=pltpu.CompilerParams(needs_layout_passes=False),
    out_type=jax.ShapeDtypeStruct((_B * _V,), jnp.float32),
    scratch_types=[
        pltpu.VMEM((_RPW * _L,), jnp.int32),
        pltpu.VMEM((_RPW * _V,), jnp.float32),
    ],
)(_sc_hist_body)


def _tc_mlp_body(h_ref, tab_ref, w1_ref, b1_ref, w2_ref, b2_ref, o_ref):
    pooled = jnp.dot(h_ref[...], tab_ref[...],
                     preferred_element_type=jnp.float32) * (1.0 / _L)
    x = jnp.dot(pooled, w1_ref[...],
                preferred_element_type=jnp.float32) + b1_ref[...]
    x = 0.5 * x * (1.0 + lax.erf(x * (1.0 / math.sqrt(2.0))))
    o_ref[...] = jnp.dot(x, w2_ref[...],
                         preferred_element_type=jnp.float32) + b2_ref[...]


def kernel(prompt_ids, table, W1, b1, W2, b2):
    h = _sc_hist(prompt_ids.astype(jnp.int32).reshape(_B * _L)).reshape(_B, _V)
    out = pl.pallas_call(
        _tc_mlp_body,
        out_shape=jax.ShapeDtypeStruct((_B, _D), jnp.float32),
    )(h, table, W1, b1.reshape(1, _D), W2, b2.reshape(1, _D))
    return out


# speedup vs baseline: 51.0086x; 1.2458x over previous
import functools
import math

import jax
import jax.numpy as jnp
from jax import lax
from jax.experimental import pallas as pl
from jax.experimental.pallas import tpu as pltpu
from jax.experimental.pallas import tpu_sc as plsc

_B, _L, _V, _D = 1024, 512, 256, 128

_NC, _NS, _LANES = 2, 16, 16
_NW = _NC * _NS
_RPW = _B // _NW
_GROUPS = _RPW // _LANES
_VCHUNKS = _V // _LANES


def _sc_hist_body(ids_hbm, h_hbm, ids_v, hist_v):
    wid = lax.axis_index("s") * _NC + lax.axis_index("c")
    base = wid * _RPW
    pltpu.sync_copy(ids_hbm.at[pl.ds(base * _L, _RPW * _L)], ids_v)

    lane = lax.iota(jnp.int32, 16)
    zeros = jnp.zeros((_LANES,), jnp.float32)
    ones = jnp.ones((_LANES,), jnp.float32)

    @plsc.parallel_loop(0, _RPW * _VCHUNKS, unroll=8)
    def _zero(i):
        hist_v[pl.ds(i * _LANES, _LANES)] = zeros

    @plsc.parallel_loop(0, _L, unroll=8)
    def _pos(p):
        for g in range(_GROUPS):
            row = g * _LANES + lane
            ids16 = plsc.load_gather(ids_v, [row * _L + p])
            plsc.addupdate_scatter(hist_v, [row * _V + ids16], ones)

    pltpu.sync_copy(hist_v, h_hbm.at[pl.ds(base * _V, _RPW * _V)])


_sc_hist = functools.partial(
    pl.kernel,
    mesh=plsc.VectorSubcoreMesh(core_axis_name="c", subcore_axis_name="s"),
    compiler_params=pltpu.CompilerParams(needs_layout_passes=False),
    out_type=jax.ShapeDtypeStruct((_B * _V,), jnp.float32),
    scratch_types=[
        pltpu.VMEM((_RPW * _L,), jnp.int32),
        pltpu.VMEM((_RPW * _V,), jnp.float32),
    ],
)(_sc_hist_body)


def _tc_mlp_body(h_ref, tab_ref, w1_ref, b1_ref, w2_ref, b2_ref, o_ref):
    pooled = jnp.dot(h_ref[...], tab_ref[...],
                     preferred_element_type=jnp.float32) * (1.0 / _L)
    x = jnp.dot(pooled, w1_ref[...],
                preferred_element_type=jnp.float32) + b1_ref[...]
    x = 0.5 * x * (1.0 + lax.erf(x * (1.0 / math.sqrt(2.0))))
    o_ref[...] = jnp.dot(x, w2_ref[...],
                         preferred_element_type=jnp.float32) + b2_ref[...]


def kernel(prompt_ids, table, W1, b1, W2, b2):
    h = _sc_hist(prompt_ids.astype(jnp.int32).reshape(_B * _L)).reshape(_B, _V)
    out = pl.pallas_call(
        _tc_mlp_body,
        out_shape=jax.ShapeDtypeStruct((_B, _D), jnp.float32),
    )(h, table, W1, b1.reshape(1, _D), W2, b2.reshape(1, _D))
    return out


# speedup vs baseline: 61.2341x; 1.2005x over previous
import functools
import math

import jax
import jax.numpy as jnp
from jax import lax
from jax.experimental import pallas as pl
from jax.experimental.pallas import tpu as pltpu
from jax.experimental.pallas import tpu_sc as plsc

_B, _L, _V, _D = 1024, 512, 256, 128

_NC, _NS, _LANES = 2, 16, 16
_NW = _NC * _NS
_RPW = _B // _NW
_GROUPS = _RPW // _LANES
_VCHUNKS = _V // _LANES


def _sc_hist_body(ids_hbm, h_hbm, ids_v, hist_v):
    wid = lax.axis_index("s") * _NC + lax.axis_index("c")
    base = wid * _RPW
    pltpu.sync_copy(ids_hbm.at[pl.ds(base * _L, _RPW * _L)], ids_v)

    lane = lax.iota(jnp.int32, 16)
    zeros = jnp.zeros((_LANES,), jnp.float32)
    ones = jnp.ones((_LANES,), jnp.float32)

    @plsc.parallel_loop(0, _RPW * _VCHUNKS, unroll=8)
    def _zero(i):
        hist_v[pl.ds(i * _LANES, _LANES)] = zeros

    @plsc.parallel_loop(0, _L, unroll=8)
    def _pos(p):
        for g in range(_GROUPS):
            row = g * _LANES + lane
            q = (p + lane) & (_L - 1)
            ids16 = plsc.load_gather(ids_v, [row * _L + q])
            plsc.addupdate_scatter(hist_v, [row * _V + ids16], ones)

    pltpu.sync_copy(hist_v, h_hbm.at[pl.ds(base * _V, _RPW * _V)])


_sc_hist = functools.partial(
    pl.kernel,
    mesh=plsc.VectorSubcoreMesh(core_axis_name="c", subcore_axis_name="s"),
    compiler_params=pltpu.CompilerParams(needs_layout_passes=False),
    out_type=jax.ShapeDtypeStruct((_B * _V,), jnp.float32),
    scratch_types=[
        pltpu.VMEM((_RPW * _L,), jnp.int32),
        pltpu.VMEM((_RPW * _V,), jnp.float32),
    ],
)(_sc_hist_body)


def _tc_mlp_body(h_ref, tab_ref, w1_ref, b1_ref, w2_ref, b2_ref, o_ref):
    pooled = jnp.dot(h_ref[...], tab_ref[...],
                     preferred_element_type=jnp.float32) * (1.0 / _L)
    x = jnp.dot(pooled, w1_ref[...],
                preferred_element_type=jnp.float32) + b1_ref[...]
    x = 0.5 * x * (1.0 + lax.erf(x * (1.0 / math.sqrt(2.0))))
    o_ref[...] = jnp.dot(x, w2_ref[...],
                         preferred_element_type=jnp.float32) + b2_ref[...]


def kernel(prompt_ids, table, W1, b1, W2, b2):
    h = _sc_hist(prompt_ids.astype(jnp.int32).reshape(_B * _L)).reshape(_B, _V)
    out = pl.pallas_call(
        _tc_mlp_body,
        out_shape=jax.ShapeDtypeStruct((_B, _D), jnp.float32),
    )(h, table, W1, b1.reshape(1, _D), W2, b2.reshape(1, _D))
    return out
